# baseline (speedup 1.0000x reference)
import functools

import jax
import jax.numpy as jnp
from jax import lax
from jax.experimental import pallas as pl
from jax.experimental.pallas import tpu as pltpu
from jax.experimental.pallas import tpu_sc as plsc

LR = 0.05
VOCAB = 1000000
BATCH = 16384

NC = 2
NS = 16
L = 16

TROWS = BATCH // NS // L
SLICE = 62496
HALF = SLICE // 4
TAIL0 = NS * SLICE
TAILN = VOCAB - TAIL0

_sc_mesh = plsc.VectorSubcoreMesh(
    core_axis_name="c", subcore_axis_name="s", num_cores=NC, num_subcores=NS)


@functools.partial(
    pl.kernel,
    out_type=[
        jax.ShapeDtypeStruct((NS * TROWS, L), jnp.float32),
        jax.ShapeDtypeStruct((VOCAB,), jnp.int32),
        jax.ShapeDtypeStruct((VOCAB,), jnp.int32),
    ],
    mesh=_sc_mesh,
    scratch_types=[
        pltpu.VMEM_SHARED((VOCAB,), jnp.int32),
        pltpu.VMEM((TROWS, L), jnp.int32),
        pltpu.VMEM((TROWS, L), jnp.int32),
        pltpu.VMEM((TROWS, L), jnp.int32),
        pltpu.VMEM((TROWS, L), jnp.int32),
        pltpu.VMEM((TROWS, L), jnp.float32),
        pltpu.VMEM((L,), jnp.int32),
        pltpu.VMEM((HALF,), jnp.int32),
        pltpu.SemaphoreType.DMA,
        pltpu.SemaphoreType.DMA,
    ],
)
def _sc_update(cs_hbm, ids_hbm, lat_hbm, gapb_hbm,
               inv_hbm, nlat_hbm, ngapb_hbm,
               tbl_sp, ids_v, lat_v, gap_v, cg_v, inv_v, cs_v, bounce_v,
               gsem, ssem):
    c = lax.axis_index("c")
    s = lax.axis_index("s")
    row0 = s * TROWS
    base = s * SLICE

    def via_bounce(src, dst, off, n):
        done = 0
        while done < n:
            m = min(HALF, n - done)
            pltpu.sync_copy(src.at[pl.ds(off + done, m)],
                            bounce_v.at[pl.ds(0, m)])
            pltpu.sync_copy(bounce_v.at[pl.ds(0, m)],
                            dst.at[pl.ds(off + done, m)])
            done += m

    pltpu.sync_copy(ids_hbm.at[pl.ds(row0, TROWS)], ids_v)
    pltpu.sync_copy(cs_hbm, cs_v)
    idxs = [ids_v[j, :] for j in range(TROWS)]

    @pl.when(c == 0)
    def _lat_core():
        via_bounce(lat_hbm, tbl_sp, base, SLICE)

        @pl.when(s == 0)
        def _tail_in():
            via_bounce(lat_hbm, tbl_sp, TAIL0, TAILN)

        plsc.subcore_barrier()
        scatters = [pltpu.async_copy(cs_v, tbl_sp.at[idxs[j]], ssem)
                    for j in range(TROWS)]
        for d in scatters:
            d.wait()
        plsc.subcore_barrier()
        via_bounce(tbl_sp, nlat_hbm, base, SLICE)

        @pl.when(s == 0)
        def _tail_out():
            via_bounce(tbl_sp, nlat_hbm, TAIL0, TAILN)

    @pl.when(c == 1)
    def _gap_core():
        lat_gathers = [pltpu.async_copy(lat_hbm.at[idxs[j]], lat_v.at[j], gsem)
                       for j in range(TROWS)]
        via_bounce(gapb_hbm, tbl_sp, base, SLICE)

        @pl.when(s == 0)
        def _tail_in():
            via_bounce(gapb_hbm, tbl_sp, TAIL0, TAILN)

        plsc.subcore_barrier()
        gap_gathers = [pltpu.async_copy(tbl_sp.at[idxs[j]], gap_v.at[j], gsem)
                       for j in range(TROWS)]
        for d in lat_gathers:
            d.wait()
        for d in gap_gathers:
            d.wait()

        cs = cs_v[...]
        for j in range(TROWS):
            lat = lat_v[j, :]
            gap = lax.bitcast_convert_type(gap_v[j, :], jnp.float32)
            coef = jnp.where(lat == 0, 1.0, LR).astype(jnp.float32)
            cg = (1.0 - LR) * gap + coef * (cs - lat).astype(jnp.float32)
            cg_v[j, :] = lax.bitcast_convert_type(cg, jnp.int32)
            inv_v[j, :] = 1.0 / cg
        pltpu.sync_copy(inv_v, inv_hbm.at[pl.ds(row0, TROWS)])

        plsc.subcore_barrier()
        scatters = [pltpu.async_copy(cg_v.at[j], tbl_sp.at[idxs[j]], ssem)
                    for j in range(TROWS)]
        for d in scatters:
            d.wait()
        plsc.subcore_barrier()
        via_bounce(tbl_sp, ngapb_hbm, base, SLICE)

        @pl.when(s == 0)
        def _tail_out():
            via_bounce(tbl_sp, ngapb_hbm, TAIL0, TAILN)


def kernel(cur_step, candidate_ids, latest_step, step_gap):
    cs16 = jnp.full((L,), cur_step, dtype=jnp.int32)
    ids2d = candidate_ids.reshape(NS * TROWS, L)
    gap_bits = lax.bitcast_convert_type(step_gap, jnp.int32)
    inv2d, new_lat, new_gap_bits = _sc_update(cs16, ids2d, latest_step,
                                              gap_bits)
    return (inv2d.reshape(BATCH),
            new_lat,
            lax.bitcast_convert_type(new_gap_bits, jnp.float32))


# speedup vs baseline: 1.0944x; 1.0944x over previous
import functools

import jax
import jax.numpy as jnp
from jax import lax
from jax.experimental import pallas as pl
from jax.experimental.pallas import tpu as pltpu
from jax.experimental.pallas import tpu_sc as plsc

LR = 0.05
VOCAB = 1000000
BATCH = 16384

NC = 2
NS = 16
L = 16

TROWS = BATCH // NS // L
SLICE = 62496
HALF = 15616
TAIL0 = NS * SLICE
TAILN = VOCAB - TAIL0

_sc_mesh = plsc.VectorSubcoreMesh(
    core_axis_name="c", subcore_axis_name="s", num_cores=NC, num_subcores=NS)


@functools.partial(
    pl.kernel,
    out_type=[
        jax.ShapeDtypeStruct((BATCH,), jnp.float32),
        jax.ShapeDtypeStruct((VOCAB,), jnp.int32),
        jax.ShapeDtypeStruct((VOCAB,), jnp.int32),
    ],
    mesh=_sc_mesh,
    scratch_types=[
        pltpu.VMEM_SHARED((VOCAB,), jnp.int32),
        pltpu.VMEM((BATCH // NS,), jnp.int32),
        pltpu.VMEM((TROWS, L), jnp.int32),
        pltpu.VMEM((TROWS, L), jnp.int32),
        pltpu.VMEM((TROWS, L), jnp.int32),
        pltpu.VMEM((BATCH // NS,), jnp.float32),
        pltpu.VMEM((L,), jnp.int32),
        pltpu.VMEM((HALF,), jnp.int32),
        pltpu.SemaphoreType.DMA,
        pltpu.SemaphoreType.DMA,
    ],
)
def _sc_update(cs_hbm, ids_hbm, lat_hbm, gapb_hbm,
               inv_hbm, nlat_hbm, ngapb_hbm,
               tbl_sp, ids_v, lat_v, gap_v, cg_v, inv_v, cs_v, bounce_v,
               gsem, ssem):
    c = lax.axis_index("c")
    s = lax.axis_index("s")
    nper = BATCH // NS
    row0 = s * nper
    base = s * SLICE
    def via_bounce(src, dst, off, n):
        done = 0
        while done < n:
            m = min(HALF, n - done)
            pltpu.sync_copy(src.at[pl.ds(off + done, m)],
                            bounce_v.at[pl.ds(0, m)])
            pltpu.sync_copy(bounce_v.at[pl.ds(0, m)],
                            dst.at[pl.ds(off + done, m)])
            done += m

    pltpu.sync_copy(ids_hbm.at[pl.ds(row0, nper)], ids_v)
    pltpu.sync_copy(cs_hbm, cs_v)
    idxs = [ids_v[pl.ds(j * L, L)] for j in range(TROWS)]

    @pl.when(c == 0)
    def _lat_core():
        via_bounce(lat_hbm, tbl_sp, base, SLICE)

        @pl.when(s == 0)
        def _tail_in():
            via_bounce(lat_hbm, tbl_sp, TAIL0, TAILN)

        plsc.subcore_barrier()
        scatters = [pltpu.async_copy(cs_v, tbl_sp.at[idxs[j]], ssem)
                    for j in range(TROWS)]
        for d in scatters:
            d.wait()
        plsc.subcore_barrier()
        via_bounce(tbl_sp, nlat_hbm, base, SLICE)

        @pl.when(s == 0)
        def _tail_out():
            via_bounce(tbl_sp, nlat_hbm, TAIL0, TAILN)

    @pl.when(c == 1)
    def _gap_core():
        lat_gathers = [pltpu.async_copy(lat_hbm.at[idxs[j]], lat_v.at[j], gsem)
                       for j in range(TROWS)]
        via_bounce(gapb_hbm, tbl_sp, base, SLICE)

        @pl.when(s == 0)
        def _tail_in():
            via_bounce(gapb_hbm, tbl_sp, TAIL0, TAILN)

        plsc.subcore_barrier()
        gap_gathers = [pltpu.async_copy(tbl_sp.at[idxs[j]], gap_v.at[j], gsem)
                       for j in range(TROWS)]
        for d in lat_gathers:
            d.wait()
        for d in gap_gathers:
            d.wait()

        cs = cs_v[...]
        for j in range(TROWS):
            lat = lat_v[j, :]
            gap = lax.bitcast_convert_type(gap_v[j, :], jnp.float32)
            coef = jnp.where(lat == 0, 1.0, LR).astype(jnp.float32)
            cg = (1.0 - LR) * gap + coef * (cs - lat).astype(jnp.float32)
            cg_v[j, :] = lax.bitcast_convert_type(cg, jnp.int32)
            inv_v[pl.ds(j * L, L)] = 1.0 / cg
        pltpu.sync_copy(inv_v, inv_hbm.at[pl.ds(row0, nper)])

        plsc.subcore_barrier()
        scatters = [pltpu.async_copy(cg_v.at[j], tbl_sp.at[idxs[j]], ssem)
                    for j in range(TROWS)]
        for d in scatters:
            d.wait()
        plsc.subcore_barrier()
        via_bounce(tbl_sp, ngapb_hbm, base, SLICE)

        @pl.when(s == 0)
        def _tail_out():
            via_bounce(tbl_sp, ngapb_hbm, TAIL0, TAILN)


def kernel(cur_step, candidate_ids, latest_step, step_gap):
    cs16 = jnp.full((L,), cur_step, dtype=jnp.int32)
    gap_bits = lax.bitcast_convert_type(step_gap, jnp.int32)
    inv, new_lat, new_gap_bits = _sc_update(cs16, candidate_ids, latest_step,
                                            gap_bits)
    return inv, new_lat, lax.bitcast_convert_type(new_gap_bits, jnp.float32)


# speedup vs baseline: 1.1961x; 1.0929x over previous
import functools

import jax
import jax.numpy as jnp
from jax import lax
from jax.experimental import pallas as pl
from jax.experimental.pallas import tpu as pltpu
from jax.experimental.pallas import tpu_sc as plsc

LR = 0.05
VOCAB = 1000000
BATCH = 16384

NC = 2
NS = 16
L = 16

TROWS = BATCH // NS // L
SLICE = 62496
HALF = 15616
TAIL0 = NS * SLICE
TAILN = VOCAB - TAIL0

_sc_mesh = plsc.VectorSubcoreMesh(
    core_axis_name="c", subcore_axis_name="s", num_cores=NC, num_subcores=NS)


@functools.partial(
    pl.kernel,
    out_type=[
        jax.ShapeDtypeStruct((BATCH,), jnp.float32),
        jax.ShapeDtypeStruct((VOCAB,), jnp.int32),
        jax.ShapeDtypeStruct((VOCAB,), jnp.int32),
    ],
    mesh=_sc_mesh,
    scratch_types=[
        pltpu.VMEM_SHARED((VOCAB,), jnp.int32),
        pltpu.VMEM((BATCH // NS,), jnp.int32),
        pltpu.VMEM((TROWS, L), jnp.int32),
        pltpu.VMEM((TROWS, L), jnp.int32),
        pltpu.VMEM((TROWS, L), jnp.int32),
        pltpu.VMEM((BATCH // NS,), jnp.float32),
        pltpu.VMEM((L,), jnp.int32),
        pltpu.VMEM((HALF,), jnp.int32),
        pltpu.VMEM((HALF,), jnp.int32),
        pltpu.SemaphoreType.DMA,
        pltpu.SemaphoreType.DMA,
        pltpu.SemaphoreType.DMA,
        pltpu.SemaphoreType.DMA,
        pltpu.SemaphoreType.DMA,
        pltpu.SemaphoreType.DMA,
    ],
)
def _sc_update(cs_hbm, ids_hbm, lat_hbm, gapb_hbm,
               inv_hbm, nlat_hbm, ngapb_hbm,
               tbl_sp, ids_v, lat_v, gap_v, cg_v, inv_v, cs_v,
               bounce_a, bounce_b,
               gsem, ssem, isem_a, isem_b, osem_a, osem_b):
    c = lax.axis_index("c")
    s = lax.axis_index("s")
    nper = BATCH // NS
    row0 = s * nper
    base = s * SLICE
    bufs = (bounce_a, bounce_b)
    isems = (isem_a, isem_b)
    osems = (osem_a, osem_b)

    def via_bounce(src, dst, off, n):
        chunks = []
        done = 0
        while done < n:
            m = min(HALF, n - done)
            chunks.append((done, m))
            done += m
        in_d = [None, None]
        out_d = [None, None]

        def start_in(k):
            o, m = chunks[k]
            b = k % 2
            in_d[b] = pltpu.async_copy(src.at[pl.ds(off + o, m)],
                                       bufs[b].at[pl.ds(0, m)], isems[b])

        start_in(0)
        for k, (o, m) in enumerate(chunks):
            b = k % 2
            if k + 1 < len(chunks):
                if out_d[1 - b] is not None:
                    out_d[1 - b].wait()
                start_in(k + 1)
            in_d[b].wait()
            out_d[b] = pltpu.async_copy(bufs[b].at[pl.ds(0, m)],
                                        dst.at[pl.ds(off + o, m)], osems[b])
        for d in out_d:
            if d is not None:
                d.wait()

    pltpu.sync_copy(ids_hbm.at[pl.ds(row0, nper)], ids_v)
    pltpu.sync_copy(cs_hbm, cs_v)
    idxs = [ids_v[pl.ds(j * L, L)] for j in range(TROWS)]

    @pl.when(c == 0)
    def _lat_core():
        via_bounce(lat_hbm, tbl_sp, base, SLICE)

        @pl.when(s == 0)
        def _tail_in():
            via_bounce(lat_hbm, tbl_sp, TAIL0, TAILN)

        plsc.subcore_barrier()
        scatters = [pltpu.async_copy(cs_v, tbl_sp.at[idxs[j]], ssem)
                    for j in range(TROWS)]
        for d in scatters:
            d.wait()
        plsc.subcore_barrier()
        via_bounce(tbl_sp, nlat_hbm, base, SLICE)

        @pl.when(s == 0)
        def _tail_out():
            via_bounce(tbl_sp, nlat_hbm, TAIL0, TAILN)

    @pl.when(c == 1)
    def _gap_core():
        lat_gathers = [pltpu.async_copy(lat_hbm.at[idxs[j]], lat_v.at[j], gsem)
                       for j in range(TROWS)]
        via_bounce(gapb_hbm, tbl_sp, base, SLICE)

        @pl.when(s == 0)
        def _tail_in():
            via_bounce(gapb_hbm, tbl_sp, TAIL0, TAILN)

        plsc.subcore_barrier()
        gap_gathers = [pltpu.async_copy(tbl_sp.at[idxs[j]], gap_v.at[j], gsem)
                       for j in range(TROWS)]
        for d in lat_gathers:
            d.wait()
        for d in gap_gathers:
            d.wait()

        cs = cs_v[...]
        for j in range(TROWS):
            lat = lat_v[j, :]
            gap = lax.bitcast_convert_type(gap_v[j, :], jnp.float32)
            coef = jnp.where(lat == 0, 1.0, LR).astype(jnp.float32)
            cg = (1.0 - LR) * gap + coef * (cs - lat).astype(jnp.float32)
            cg_v[j, :] = lax.bitcast_convert_type(cg, jnp.int32)
            inv_v[pl.ds(j * L, L)] = 1.0 / cg
        pltpu.sync_copy(inv_v, inv_hbm.at[pl.ds(row0, nper)])

        plsc.subcore_barrier()
        scatters = [pltpu.async_copy(cg_v.at[j], tbl_sp.at[idxs[j]], ssem)
                    for j in range(TROWS)]
        for d in scatters:
            d.wait()
        plsc.subcore_barrier()
        via_bounce(tbl_sp, ngapb_hbm, base, SLICE)

        @pl.when(s == 0)
        def _tail_out():
            via_bounce(tbl_sp, ngapb_hbm, TAIL0, TAILN)


def kernel(cur_step, candidate_ids, latest_step, step_gap):
    cs16 = jnp.full((L,), cur_step, dtype=jnp.int32)
    gap_bits = lax.bitcast_convert_type(step_gap, jnp.int32)
    inv, new_lat, new_gap_bits = _sc_update(cs16, candidate_ids, latest_step,
                                            gap_bits)
    return inv, new_lat, lax.bitcast_convert_type(new_gap_bits, jnp.float32)
